# baseline (speedup 1.0000x reference)
import functools

import jax
import jax.numpy as jnp
from jax.experimental import pallas as pl

N_CTRL = 32


def _body(table_ref, out_ref):
    out_ref[...] = jnp.broadcast_to(table_ref[...][None], out_ref.shape)


@functools.cache
def _make_kernel(B, D):
    return pl.pallas_call(
        _body,
        grid=(2,),
        in_specs=[pl.BlockSpec((N_CTRL, D), lambda g: (0, 0))],
        out_specs=pl.BlockSpec((B // 2, N_CTRL, D), lambda g: (g, 0, 0)),
        out_shape=jax.ShapeDtypeStruct((B, N_CTRL, D), jnp.float32),
    )


def kernel(x, embed_table):
    B = x.shape[0]
    D = embed_table.shape[1]
    return _make_kernel(B, D)(embed_table)
